# Initial kernel scaffold; baseline (speedup 1.0000x reference)
import jax
import jax.numpy as jnp
from jax.experimental import pallas as pl


def kernel(x, batch):
    raise NotImplementedError("write your pallas kernel here")


# speedup vs baseline: 3.8743x; 3.8743x over previous
import jax
import jax.numpy as jnp
from jax import lax
from jax.experimental import pallas as pl
from jax.experimental.pallas import tpu as pltpu
from jax.experimental.pallas import tpu_sc as plsc

NUM_ROWS = 320000
D = 128
NUM_SEG = 10000
NUM_CORES = 2
NUM_SUBCORES = 16
NW = NUM_CORES * NUM_SUBCORES
SEG_PER_W = 320
SEG_PER_CORE = NUM_SUBCORES * SEG_PER_W
ACC_ROWS = 5128
DUMP_ROW = 5120
CHUNK = 128
LAST_W_SEGS = NUM_SEG - SEG_PER_W * (NW - 1)


def _seg_sum_body(x_hbm, batch_hbm, bounds_hbm, out_hbm, xbuf, idxbuf, bvm, acc):
    c = lax.axis_index("c")
    s = lax.axis_index("s")
    w = c * NUM_SUBCORES + s
    iota = lax.iota(jnp.int32, 16)

    def zero_row(r, carry):
        for j in range(D // 16):
            xbuf[r, pl.ds(j * 16, 16)] = jnp.zeros((16,), jnp.float32)
        return carry

    lax.fori_loop(0, CHUNK, zero_row, 0)
    region = SEG_PER_W * s
    pltpu.sync_copy(xbuf, acc.at[pl.ds(region, CHUNK)])
    pltpu.sync_copy(xbuf, acc.at[pl.ds(region + CHUNK, CHUNK)])
    pltpu.sync_copy(
        xbuf.at[pl.ds(0, SEG_PER_W - 2 * CHUNK)],
        acc.at[pl.ds(region + 2 * CHUNK, SEG_PER_W - 2 * CHUNK)],
    )

    pltpu.sync_copy(bounds_hbm, bvm)
    lo = bvm[pl.ds(w, 16)][0]
    hi = bvm[pl.ds(w + 1, 16)][0]

    seg_base = c * SEG_PER_CORE
    aligned_lo = jnp.bitwise_and(lo, -8)
    num_chunks = (hi - aligned_lo + CHUNK - 1) // CHUNK

    def chunk_body(k, carry):
        start = aligned_lo + k * CHUNK
        q = pl.multiple_of(jnp.minimum(start, NUM_ROWS - CHUNK), 8)
        lo_k = jnp.maximum(lo, start)
        hi_k = jnp.minimum(hi, start + CHUNK)
        pltpu.sync_copy(x_hbm.at[pl.ds(q, CHUNK)], xbuf)
        pltpu.sync_copy(batch_hbm.at[pl.ds(q, CHUNK)], idxbuf)
        for j in range(CHUNK // 16):
            iv = idxbuf[pl.ds(j * 16, 16)]
            rowid = q + j * 16 + iota
            valid = (rowid >= lo_k) & (rowid < hi_k)
            idxbuf[pl.ds(j * 16, 16)] = jnp.where(valid, iv - seg_base, DUMP_ROW)
        pltpu.sync_copy(xbuf, acc.at[idxbuf], add=True)
        return carry

    lax.fori_loop(0, num_chunks, chunk_body, 0)

    out_off = SEG_PER_W * w

    @pl.when(w == NW - 1)
    def _():
        pltpu.sync_copy(
            acc.at[pl.ds(region, LAST_W_SEGS)], out_hbm.at[pl.ds(out_off, LAST_W_SEGS)]
        )

    @pl.when(w != NW - 1)
    def _():
        pltpu.sync_copy(
            acc.at[pl.ds(region, SEG_PER_W)], out_hbm.at[pl.ds(out_off, SEG_PER_W)]
        )


@jax.jit
def kernel(x, batch):
    batch = batch.astype(jnp.int32)
    seg_edges = jnp.arange(NW + 1, dtype=jnp.int32) * SEG_PER_W
    bounds = jnp.searchsorted(batch, seg_edges, side="left").astype(jnp.int32)
    bounds = jnp.concatenate(
        [bounds, jnp.full((48 - (NW + 1),), NUM_ROWS, jnp.int32)]
    )

    mesh = plsc.VectorSubcoreMesh(core_axis_name="c", subcore_axis_name="s")
    fn = pl.kernel(
        _seg_sum_body,
        mesh=mesh,
        out_type=jax.ShapeDtypeStruct((NUM_SEG, D), jnp.float32),
        scratch_types=[
            pltpu.VMEM((CHUNK, D), jnp.float32),
            pltpu.VMEM((CHUNK,), jnp.int32),
            pltpu.VMEM((48,), jnp.int32),
            pltpu.VMEM_SHARED((ACC_ROWS, D), jnp.float32),
        ],
    )
    return fn(x, batch, bounds)


# speedup vs baseline: 6.1739x; 1.5935x over previous
import jax
import jax.numpy as jnp
from jax import lax
from jax.experimental import pallas as pl
from jax.experimental.pallas import tpu as pltpu
from jax.experimental.pallas import tpu_sc as plsc

NUM_ROWS = 320000
D = 128
NUM_SEG = 10000
NUM_CORES = 2
NUM_SUBCORES = 16
NW = NUM_CORES * NUM_SUBCORES
SEG_PER_W = 320
SEG_PER_CORE = NUM_SUBCORES * SEG_PER_W
ACC_ROWS = 5128
DUMP_ROW = 5120
CHUNK = 128
LAST_W_SEGS = NUM_SEG - SEG_PER_W * (NW - 1)


def _seg_sum_body(
    x_hbm, batch_hbm, bounds_hbm, out_hbm,
    xbuf0, xbuf1, ibuf0, ibuf1, bvm, acc,
    semx0, semx1, semi0, semi1,
):
    c = lax.axis_index("c")
    s = lax.axis_index("s")
    w = c * NUM_SUBCORES + s
    iota = lax.iota(jnp.int32, 16)
    xbufs = (xbuf0, xbuf1)
    ibufs = (ibuf0, ibuf1)
    semxs = (semx0, semx1)
    semis = (semi0, semi1)

    def zero_row(r, carry):
        for j in range(D // 16):
            xbuf0[r, pl.ds(j * 16, 16)] = jnp.zeros((16,), jnp.float32)
        return carry

    lax.fori_loop(0, CHUNK, zero_row, 0)
    region = SEG_PER_W * s
    pltpu.sync_copy(xbuf0, acc.at[pl.ds(region, CHUNK)])
    pltpu.sync_copy(xbuf0, acc.at[pl.ds(region + CHUNK, CHUNK)])
    pltpu.sync_copy(
        xbuf0.at[pl.ds(0, SEG_PER_W - 2 * CHUNK)],
        acc.at[pl.ds(region + 2 * CHUNK, SEG_PER_W - 2 * CHUNK)],
    )

    pltpu.sync_copy(bounds_hbm, bvm)
    lo = bvm[pl.ds(w, 16)][0]
    hi = bvm[pl.ds(w + 1, 16)][0]

    seg_base = c * SEG_PER_CORE
    aligned_lo = jnp.bitwise_and(lo, -8)
    num_chunks = (hi - aligned_lo + CHUNK - 1) // CHUNK

    def chunk_q(k):
        start = aligned_lo + k * CHUNK
        return pl.multiple_of(jnp.minimum(start, NUM_ROWS - CHUNK), 8)

    def issue_load(k, p):
        q = chunk_q(k)
        pltpu.async_copy(x_hbm.at[pl.ds(q, CHUNK)], xbufs[p], semxs[p])
        pltpu.async_copy(batch_hbm.at[pl.ds(q, CHUNK)], ibufs[p], semis[p])

    @pl.when(num_chunks > 0)
    def _():
        issue_load(0, 0)

    def chunk_iter(k, b):
        @pl.when(k < num_chunks)
        def _():
            @pl.when(k + 1 < num_chunks)
            def _():
                issue_load(k + 1, 1 - b)

            start = aligned_lo + k * CHUNK
            q = chunk_q(k)
            lo_k = jnp.maximum(lo, start)
            hi_k = jnp.minimum(hi, start + CHUNK)
            pltpu.make_async_copy(
                batch_hbm.at[pl.ds(q, CHUNK)], ibufs[b], semis[b]
            ).wait()
            for j in range(CHUNK // 16):
                iv = ibufs[b][pl.ds(j * 16, 16)]
                rowid = q + j * 16 + iota
                valid = (rowid >= lo_k) & (rowid < hi_k)
                ibufs[b][pl.ds(j * 16, 16)] = jnp.where(
                    valid, iv - seg_base, DUMP_ROW
                )
            pltpu.make_async_copy(
                x_hbm.at[pl.ds(q, CHUNK)], xbufs[b], semxs[b]
            ).wait()
            pltpu.sync_copy(xbufs[b], acc.at[ibufs[b]], add=True)

    def outer(g, carry):
        chunk_iter(2 * g, 0)
        chunk_iter(2 * g + 1, 1)
        return carry

    lax.fori_loop(0, (num_chunks + 1) // 2, outer, 0)

    out_off = SEG_PER_W * w

    @pl.when(w == NW - 1)
    def _():
        pltpu.sync_copy(
            acc.at[pl.ds(region, LAST_W_SEGS)], out_hbm.at[pl.ds(out_off, LAST_W_SEGS)]
        )

    @pl.when(w != NW - 1)
    def _():
        pltpu.sync_copy(
            acc.at[pl.ds(region, SEG_PER_W)], out_hbm.at[pl.ds(out_off, SEG_PER_W)]
        )


@jax.jit
def kernel(x, batch):
    batch = batch.astype(jnp.int32)
    seg_edges = jnp.arange(NW + 1, dtype=jnp.int32) * SEG_PER_W
    bounds = jnp.searchsorted(batch, seg_edges, side="left").astype(jnp.int32)
    bounds = jnp.concatenate(
        [bounds, jnp.full((48 - (NW + 1),), NUM_ROWS, jnp.int32)]
    )

    mesh = plsc.VectorSubcoreMesh(core_axis_name="c", subcore_axis_name="s")
    fn = pl.kernel(
        _seg_sum_body,
        mesh=mesh,
        out_type=jax.ShapeDtypeStruct((NUM_SEG, D), jnp.float32),
        scratch_types=[
            pltpu.VMEM((CHUNK, D), jnp.float32),
            pltpu.VMEM((CHUNK, D), jnp.float32),
            pltpu.VMEM((CHUNK,), jnp.int32),
            pltpu.VMEM((CHUNK,), jnp.int32),
            pltpu.VMEM((48,), jnp.int32),
            pltpu.VMEM_SHARED((ACC_ROWS, D), jnp.float32),
            pltpu.SemaphoreType.DMA,
            pltpu.SemaphoreType.DMA,
            pltpu.SemaphoreType.DMA,
            pltpu.SemaphoreType.DMA,
        ],
    )
    return fn(x, batch, bounds)


# speedup vs baseline: 7.7775x; 1.2598x over previous
import jax
import jax.numpy as jnp
from jax import lax
from jax.experimental import pallas as pl
from jax.experimental.pallas import tpu as pltpu
from jax.experimental.pallas import tpu_sc as plsc

NUM_ROWS = 320000
D = 128
NUM_SEG = 10000
NUM_CORES = 2
NUM_SUBCORES = 16
NW = NUM_CORES * NUM_SUBCORES
SEG_PER_W = 320
SEG_PER_CORE = NUM_SUBCORES * SEG_PER_W
ACC_ROWS = 5128
DUMP_ROW = 5120
CHUNK = 128
LAST_W_SEGS = NUM_SEG - SEG_PER_W * (NW - 1)


def _seg_sum_body(
    x_hbm, batch_hbm, bounds_hbm, out_hbm,
    xbuf0, xbuf1, ibuf0, ibuf1, bvm, acc,
    semx0, semx1, semi0, semi1,
):
    c = lax.axis_index("c")
    s = lax.axis_index("s")
    w = c * NUM_SUBCORES + s
    iota = lax.iota(jnp.int32, 16)
    xbufs = (xbuf0, xbuf1)
    ibufs = (ibuf0, ibuf1)
    semxs = (semx0, semx1)
    semis = (semi0, semi1)

    def zero_row(r, carry):
        for j in range(D // 16):
            xbuf0[r, pl.ds(j * 16, 16)] = jnp.zeros((16,), jnp.float32)
        return carry

    lax.fori_loop(0, CHUNK, zero_row, 0)
    region = SEG_PER_W * s
    pltpu.sync_copy(xbuf0, acc.at[pl.ds(region, CHUNK)])
    pltpu.sync_copy(xbuf0, acc.at[pl.ds(region + CHUNK, CHUNK)])
    pltpu.sync_copy(
        xbuf0.at[pl.ds(0, SEG_PER_W - 2 * CHUNK)],
        acc.at[pl.ds(region + 2 * CHUNK, SEG_PER_W - 2 * CHUNK)],
    )

    pltpu.sync_copy(bounds_hbm, bvm)
    lo = bvm[pl.ds(w, 16)][0]
    hi = bvm[pl.ds(w + 1, 16)][0]

    seg_base = c * SEG_PER_CORE
    aligned_lo = jnp.bitwise_and(lo, -8)
    num_chunks = (hi - aligned_lo + CHUNK - 1) // CHUNK

    def chunk_q(k):
        start = aligned_lo + k * CHUNK
        return pl.multiple_of(jnp.minimum(start, NUM_ROWS - CHUNK), 8)

    def issue_load(k, p):
        q = chunk_q(k)
        pltpu.async_copy(x_hbm.at[pl.ds(q, CHUNK)], xbufs[p], semxs[p])
        pltpu.async_copy(batch_hbm.at[pl.ds(q, CHUNK)], ibufs[p], semis[p])

    @pl.when(num_chunks > 0)
    def _():
        issue_load(0, 0)

    def chunk_iter(k, b):
        @pl.when(k < num_chunks)
        def _():
            @pl.when(k + 1 < num_chunks)
            def _():
                issue_load(k + 1, 1 - b)

            start = aligned_lo + k * CHUNK
            q = chunk_q(k)
            lo_k = jnp.maximum(lo, start)
            hi_k = jnp.minimum(hi, start + CHUNK)
            pltpu.make_async_copy(
                batch_hbm.at[pl.ds(q, CHUNK)], ibufs[b], semis[b]
            ).wait()
            for j in range(CHUNK // 16):
                iv = ibufs[b][pl.ds(j * 16, 16)]
                rowid = q + j * 16 + iota
                valid = (rowid >= lo_k) & (rowid < hi_k)
                ibufs[b][pl.ds(j * 16, 16)] = jnp.where(
                    valid, iv - seg_base, DUMP_ROW
                )
            pltpu.make_async_copy(
                x_hbm.at[pl.ds(q, CHUNK)], xbufs[b], semxs[b]
            ).wait()
            pltpu.sync_copy(xbufs[b], acc.at[ibufs[b]], add=True)

    def outer(g, carry):
        chunk_iter(2 * g, 0)
        chunk_iter(2 * g + 1, 1)
        return carry

    lax.fori_loop(0, (num_chunks + 1) // 2, outer, 0)

    out_off = SEG_PER_W * w

    @pl.when(w == NW - 1)
    def _():
        pltpu.sync_copy(
            acc.at[pl.ds(region, LAST_W_SEGS)], out_hbm.at[pl.ds(out_off, LAST_W_SEGS)]
        )

    @pl.when(w != NW - 1)
    def _():
        pltpu.sync_copy(
            acc.at[pl.ds(region, SEG_PER_W)], out_hbm.at[pl.ds(out_off, SEG_PER_W)]
        )


@jax.jit
def kernel(x, batch):
    batch = batch.astype(jnp.int32)
    seg_edges = jnp.arange(NW + 1, dtype=jnp.int32) * SEG_PER_W
    bounds = jnp.searchsorted(
        batch, seg_edges, side="left", method="scan_unrolled"
    ).astype(jnp.int32)
    bounds = jnp.concatenate(
        [bounds, jnp.full((48 - (NW + 1),), NUM_ROWS, jnp.int32)]
    )

    mesh = plsc.VectorSubcoreMesh(core_axis_name="c", subcore_axis_name="s")
    fn = pl.kernel(
        _seg_sum_body,
        mesh=mesh,
        out_type=jax.ShapeDtypeStruct((NUM_SEG, D), jnp.float32),
        scratch_types=[
            pltpu.VMEM((CHUNK, D), jnp.float32),
            pltpu.VMEM((CHUNK, D), jnp.float32),
            pltpu.VMEM((CHUNK,), jnp.int32),
            pltpu.VMEM((CHUNK,), jnp.int32),
            pltpu.VMEM((48,), jnp.int32),
            pltpu.VMEM_SHARED((ACC_ROWS, D), jnp.float32),
            pltpu.SemaphoreType.DMA,
            pltpu.SemaphoreType.DMA,
            pltpu.SemaphoreType.DMA,
            pltpu.SemaphoreType.DMA,
        ],
    )
    return fn(x, batch, bounds)


# speedup vs baseline: 7.8831x; 1.0136x over previous
import jax
import jax.numpy as jnp
from jax import lax
from jax.experimental import pallas as pl
from jax.experimental.pallas import tpu as pltpu
from jax.experimental.pallas import tpu_sc as plsc

NUM_ROWS = 320000
D = 128
NUM_SEG = 10000
NUM_CORES = 2
NUM_SUBCORES = 16
CHUNK = 128
WINDOWS_PER_CORE = NUM_ROWS // (NUM_CORES * CHUNK)
ACC_ROWS = 10112
ZROWS = ACC_ROWS // NUM_SUBCORES
NRING = 3


def _phase1_body(
    x_hbm, batch_hbm, zeros_hbm, a0_hbm, a1_hbm,
    xbuf0, xbuf1, xbuf2, ibuf0, ibuf1, ibuf2, acc,
    semx0, semx1, semx2, semi0, semi1, semi2,
):
    c = lax.axis_index("c")
    s = lax.axis_index("s")
    xbufs = (xbuf0, xbuf1, xbuf2)
    ibufs = (ibuf0, ibuf1, ibuf2)
    semxs = (semx0, semx1, semx2)
    semis = (semi0, semi1, semi2)

    pltpu.sync_copy(zeros_hbm, xbuf0)
    for z in range(ZROWS // CHUNK):
        pltpu.sync_copy(xbuf0, acc.at[pl.ds(ZROWS * s + z * CHUNK, CHUNK)])
    rem = ZROWS % CHUNK
    if rem:
        pltpu.sync_copy(
            xbuf0.at[pl.ds(0, rem)],
            acc.at[pl.ds(ZROWS * s + ZROWS - rem, rem)],
        )
    plsc.subcore_barrier()

    num_windows = (WINDOWS_PER_CORE - s + NUM_SUBCORES - 1) // NUM_SUBCORES
    row0 = c * (WINDOWS_PER_CORE * CHUNK)

    def window_q(t):
        return pl.multiple_of(row0 + (s + t * NUM_SUBCORES) * CHUNK, 8)

    def issue_load(t, b):
        q = window_q(t)
        pltpu.async_copy(x_hbm.at[pl.ds(q, CHUNK)], xbufs[b], semxs[b])
        pltpu.async_copy(batch_hbm.at[pl.ds(q, CHUNK)], ibufs[b], semis[b])

    for b in range(NRING - 1):
        @pl.when(b < num_windows)
        def _():
            issue_load(b, b)

    def tri(g, carry):
        for b in range(NRING):
            t = NRING * g + b

            @pl.when(t < num_windows)
            def _():
                @pl.when(t + NRING - 1 < num_windows)
                def _():
                    issue_load(t + NRING - 1, (b + NRING - 1) % NRING)

                q = window_q(t)
                pltpu.make_async_copy(
                    batch_hbm.at[pl.ds(q, CHUNK)], ibufs[b], semis[b]
                ).wait()
                pltpu.make_async_copy(
                    x_hbm.at[pl.ds(q, CHUNK)], xbufs[b], semxs[b]
                ).wait()
                pltpu.sync_copy(xbufs[b], acc.at[ibufs[b]], add=True)

        return carry

    lax.fori_loop(0, (num_windows + NRING - 1) // NRING, tri, 0)
    plsc.subcore_barrier()

    @pl.when(c == 0)
    def _():
        pltpu.sync_copy(
            acc.at[pl.ds(ZROWS * s, ZROWS)], a0_hbm.at[pl.ds(ZROWS * s, ZROWS)]
        )

    @pl.when(c == 1)
    def _():
        pltpu.sync_copy(
            acc.at[pl.ds(ZROWS * s, ZROWS)], a1_hbm.at[pl.ds(ZROWS * s, ZROWS)]
        )


def _add_body(a_ref, b_ref, o_ref):
    o_ref[...] = a_ref[...] + b_ref[...]


@jax.jit
def kernel(x, batch):
    batch = batch.astype(jnp.int32)
    zeros = jnp.zeros((CHUNK, D), jnp.float32)

    mesh = plsc.VectorSubcoreMesh(core_axis_name="c", subcore_axis_name="s")
    phase1 = pl.kernel(
        _phase1_body,
        mesh=mesh,
        out_type=(
            jax.ShapeDtypeStruct((ACC_ROWS, D), jnp.float32),
            jax.ShapeDtypeStruct((ACC_ROWS, D), jnp.float32),
        ),
        scratch_types=[
            pltpu.VMEM((CHUNK, D), jnp.float32),
            pltpu.VMEM((CHUNK, D), jnp.float32),
            pltpu.VMEM((CHUNK, D), jnp.float32),
            pltpu.VMEM((CHUNK,), jnp.int32),
            pltpu.VMEM((CHUNK,), jnp.int32),
            pltpu.VMEM((CHUNK,), jnp.int32),
            pltpu.VMEM_SHARED((ACC_ROWS, D), jnp.float32),
            pltpu.SemaphoreType.DMA,
            pltpu.SemaphoreType.DMA,
            pltpu.SemaphoreType.DMA,
            pltpu.SemaphoreType.DMA,
            pltpu.SemaphoreType.DMA,
            pltpu.SemaphoreType.DMA,
        ],
    )
    a0, a1 = phase1(x, batch, zeros)

    blk = 1000
    out = pl.pallas_call(
        _add_body,
        grid=(NUM_SEG // blk,),
        in_specs=[
            pl.BlockSpec((blk, D), lambda i: (i, 0)),
            pl.BlockSpec((blk, D), lambda i: (i, 0)),
        ],
        out_specs=pl.BlockSpec((blk, D), lambda i: (i, 0)),
        out_shape=jax.ShapeDtypeStruct((NUM_SEG, D), jnp.float32),
    )(a0, a1)
    return out
